# baseline (speedup 1.0000x reference)
import functools

import jax
import jax.numpy as jnp
from jax import lax
from jax.experimental import pallas as pl
from jax.experimental.pallas import tpu as pltpu
from jax.experimental.pallas import tpu_sc as plsc

_B = 32
_F = 3
_P = 1024
_R = _B * _F
_N2 = int(0.9 * _P)
_NV = _P // 16


def _threefry2x32(ks0, ks1, x0, x1):
    ks2 = ks0 ^ ks1 ^ jnp.int32(0x1BD11BDA)
    ks = (ks0, ks1, ks2)
    rots = ((13, 15, 26, 6), (17, 29, 16, 24))
    x0 = x0 + ks0
    x1 = x1 + ks1
    for g in range(5):
        for r in rots[g % 2]:
            x0 = x0 + x1
            x1 = (x1 << jnp.int32(r)) | lax.shift_right_logical(x1, jnp.int32(32 - r))
            x1 = x1 ^ x0
        x0 = x0 + ks[(g + 1) % 3]
        x1 = x1 + ks[(g + 2) % 3] + jnp.int32(g + 1)
    return x0, x1


def _sc_body(out_hbm, m_v, o_v):
    cid = lax.axis_index("c")
    sid = lax.axis_index("s")
    w = sid * 2 + cid

    zero = jnp.int32(0)
    k1h, k1l = _threefry2x32(zero, jnp.int32(42), zero, zero)
    k2h, k2l = _threefry2x32(zero, jnp.int32(42), zero, jnp.int32(1))
    u0, u1 = _threefry2x32(k1h, k1l, zero, w.astype(jnp.int32))
    n1 = lax.shift_right_logical(u0 ^ u1, jnp.int32(22))

    def row_body(k, _):
        r = 3 * w + k
        n = jnp.where(k == 0, n1, jnp.where(k == 1, jnp.int32(_N2), jnp.int32(_P) - n1))

        def gen(v, _c):
            lane = lax.iota(jnp.int32, 16)
            cnt = r * jnp.int32(_P) + v * jnp.int32(16) + lane
            y0, y1 = _threefry2x32(k2h, k2l, zero, cnt)
            m_v[pl.ds(v * 16, 16)] = lax.shift_right_logical(y0 ^ y1, jnp.int32(9))
            return _c
        lax.fori_loop(0, _NV, gen, zero, unroll=4)

        pref = zero
        rem = n
        for bit in range(22, -1, -1):
            tgt = pref << jnp.int32(1)

            def cpass(v, acc, _bit=bit, _tgt=tgt):
                mv = m_v[pl.ds(v * 16, 16)]
                hit = lax.shift_right_logical(mv, jnp.int32(_bit)) == _tgt
                return acc + jnp.sum(hit.astype(jnp.int32))
            c0 = lax.fori_loop(0, _NV, cpass, zero, unroll=8)
            go1 = rem > c0
            pref = tgt | go1.astype(jnp.int32)
            rem = rem - jnp.where(go1, c0, zero)
        t = pref

        prefj = zero
        remj = rem
        for bit in range(9, -1, -1):
            tgtj = prefj << jnp.int32(1)

            def jpass(v, acc, _bit=bit, _tgtj=tgtj):
                mv = m_v[pl.ds(v * 16, 16)]
                jj = v * jnp.int32(16) + lax.iota(jnp.int32, 16)
                hit = (mv == t) & (lax.shift_right_logical(jj, jnp.int32(_bit)) == _tgtj)
                return acc + jnp.sum(hit.astype(jnp.int32))
            c0 = lax.fori_loop(0, _NV, jpass, zero, unroll=8)
            go1 = remj > c0
            prefj = tgtj | go1.astype(jnp.int32)
            remj = remj - jnp.where(go1, c0, zero)
        t_eff = jnp.where(n > 0, t, jnp.int32(-1))
        j_eff = jnp.where(n > 0, prefj, jnp.int32(-1))

        def emit(v, _c):
            mv = m_v[pl.ds(v * 16, 16)]
            jj = v * jnp.int32(16) + lax.iota(jnp.int32, 16)
            mask = (mv < t_eff) | ((mv == t_eff) & (jj <= j_eff))
            o_v[pl.ds(v * 16, 16)] = mask.astype(jnp.int32)
            return _c
        lax.fori_loop(0, _NV, emit, zero, unroll=8)
        pltpu.sync_copy(o_v, out_hbm.at[pl.ds(r * _P, _P)])
        return zero

    lax.fori_loop(0, 3, row_body, zero)


def kernel(x):
    sc_fn = functools.partial(
        pl.kernel,
        out_type=jax.ShapeDtypeStruct((_R * _P,), jnp.int32),
        mesh=plsc.VectorSubcoreMesh(core_axis_name="c", subcore_axis_name="s"),
        compiler_params=pltpu.CompilerParams(needs_layout_passes=False),
        scratch_types=[
            pltpu.VMEM((_P,), jnp.int32),
            pltpu.VMEM((_P,), jnp.int32),
        ],
    )(_sc_body)
    flat = sc_fn()
    return flat.reshape(_B, _F * _P).astype(jnp.bool_)
